# Initial kernel scaffold; baseline (speedup 1.0000x reference)
import functools
import jax
import jax.numpy as jnp
from jax import lax
from jax.experimental import pallas as pl
from jax.experimental.pallas import tpu as pltpu
from jax.experimental.pallas import tpu_sc as plsc

N_ATOMS = 10000
N_NEI = 32
N_EDGES = N_ATOMS * N_NEI
NW = 32
EPT = N_EDGES // NW
NPAD = 10240
SLC = NPAD // 16
BATOMS = 1000
GRID = N_ATOMS // BATOMS

_SC_MESH = plsc.VectorSubcoreMesh(core_axis_name="c", subcore_axis_name="s")


def _gather_body(cx, cy, cz, nidx, ox, oy, oz, idx_v, vx, vy, vz, sem):
    wid = lax.axis_index("s") * 2 + lax.axis_index("c")
    base = wid * EPT
    pltpu.sync_copy(nidx.at[pl.ds(base, EPT)], idx_v)
    pltpu.async_copy(cx.at[idx_v], vx, sem).wait()
    pltpu.async_copy(cy.at[idx_v], vy, sem).wait()
    pltpu.async_copy(cz.at[idx_v], vz, sem).wait()
    pltpu.sync_copy(vx, ox.at[pl.ds(base, EPT)])
    pltpu.sync_copy(vy, oy.at[pl.ds(base, EPT)])
    pltpu.sync_copy(vz, oz.at[pl.ds(base, EPT)])


_sc_gather = pl.kernel(
    _gather_body,
    out_type=[jax.ShapeDtypeStruct((N_EDGES,), jnp.float32)] * 3,
    mesh=_SC_MESH,
    scratch_types=[
        pltpu.VMEM((EPT,), jnp.int32),
        pltpu.VMEM((EPT,), jnp.float32),
        pltpu.VMEM((EPT,), jnp.float32),
        pltpu.VMEM((EPT,), jnp.float32),
        pltpu.SemaphoreType.DMA,
    ],
)


def _scatter_body(fex, fey, fez, nidx, px, py, pz,
                  idx_v, val_v, zbuf, accx, accy, accz, sem):
    c = lax.axis_index("c")
    s = lax.axis_index("s")
    wid = s * 2 + c
    base = wid * EPT

    def _z(i, carry):
        zbuf[pl.ds(i * 16, 16)] = jnp.zeros((16,), jnp.float32)
        return carry
    lax.fori_loop(0, SLC // 16, _z, 0)
    pltpu.sync_copy(zbuf, accx.at[pl.ds(s * SLC, SLC)])
    pltpu.sync_copy(zbuf, accy.at[pl.ds(s * SLC, SLC)])
    pltpu.sync_copy(zbuf, accz.at[pl.ds(s * SLC, SLC)])
    plsc.subcore_barrier()

    pltpu.sync_copy(nidx.at[pl.ds(base, EPT)], idx_v)
    pltpu.sync_copy(fex.at[pl.ds(base, EPT)], val_v)
    pltpu.sync_copy(val_v, accx.at[idx_v], add=True)
    pltpu.sync_copy(fey.at[pl.ds(base, EPT)], val_v)
    pltpu.sync_copy(val_v, accy.at[idx_v], add=True)
    pltpu.sync_copy(fez.at[pl.ds(base, EPT)], val_v)
    pltpu.sync_copy(val_v, accz.at[idx_v], add=True)
    plsc.subcore_barrier()

    pltpu.sync_copy(accx.at[pl.ds(s * SLC, SLC)], px.at[c, pl.ds(s * SLC, SLC)])
    pltpu.sync_copy(accy.at[pl.ds(s * SLC, SLC)], py.at[c, pl.ds(s * SLC, SLC)])
    pltpu.sync_copy(accz.at[pl.ds(s * SLC, SLC)], pz.at[c, pl.ds(s * SLC, SLC)])


_sc_scatter = pl.kernel(
    _scatter_body,
    out_type=[jax.ShapeDtypeStruct((2, NPAD), jnp.float32)] * 3,
    mesh=_SC_MESH,
    scratch_types=[
        pltpu.VMEM((EPT,), jnp.int32),
        pltpu.VMEM((EPT,), jnp.float32),
        pltpu.VMEM((SLC,), jnp.float32),
        pltpu.VMEM_SHARED((NPAD,), jnp.float32),
        pltpu.VMEM_SHARED((NPAD,), jnp.float32),
        pltpu.VMEM_SHARED((NPAD,), jnp.float32),
        pltpu.SemaphoreType.DMA,
    ],
)


def _dense_body(nbx, nby, nbz, cc, at, avg, istd,
                w0, b0, w1, b1, w2, b2, w3, b3, w4, b4, w5r, b5,
                w1t, w2t, w3t, w4t, w0t,
                fex, fey, fez, fself, ener):
    i = pl.program_id(0)
    B = BATOMS
    dot = functools.partial(jnp.dot, precision=lax.Precision.HIGHEST,
                            preferred_element_type=jnp.float32)

    cxc = cc[:, 0:1]
    cyc = cc[:, 1:2]
    czc = cc[:, 2:3]
    rx = nbx[...] - cxc
    ry = nby[...] - cyc
    rz = nbz[...] - czc
    r2 = rx * rx + ry * ry + rz * rz + 1e-6
    inv_r2 = 1.0 / r2
    r = jnp.sqrt(r2)
    inv_r = 1.0 / r

    raw = jnp.concatenate([inv_r, rx * inv_r2, ry * inv_r2, rz * inv_r2],
                          axis=1)
    sel = jnp.broadcast_to(at[:, 0:1] == 0, (B, 128))
    avg_row = jnp.where(sel, jnp.broadcast_to(avg[0:1, :], (B, 128)),
                        jnp.broadcast_to(avg[1:2, :], (B, 128)))
    istd_row = jnp.where(sel, jnp.broadcast_to(istd[0:1, :], (B, 128)),
                         jnp.broadcast_to(istd[1:2, :], (B, 128)))
    sdesc = (raw - avg_row) * istd_row

    h0 = jnp.tanh(dot(sdesc, w0[...]) + b0[...])
    h1 = jnp.tanh(dot(h0, w1[...]) + b1[...])
    h2 = jnp.tanh(dot(h1, w2[...]) + b2[...])
    h3 = jnp.tanh(dot(h2, w3[...]) + b3[...])
    h4 = jnp.tanh(dot(h3, w4[...]) + b4[...])
    atom_e = jnp.sum(h4 * w5r[...], axis=1, keepdims=True) + b5[0:1, 0:1]

    @pl.when(i == 0)
    def _():
        ener[0, 0] = 0.0
    ener[0, 0] += jnp.sum(atom_e)

    d4 = (1.0 - h4 * h4) * w5r[...]
    d3 = dot(d4, w4t[...]) * (1.0 - h3 * h3)
    d2 = dot(d3, w3t[...]) * (1.0 - h2 * h2)
    d1 = dot(d2, w2t[...]) * (1.0 - h1 * h1)
    d0 = dot(d1, w1t[...]) * (1.0 - h0 * h0)
    g = dot(d0, w0t[...]) * istd_row

    g0 = g[:, 0:32]
    gx = g[:, 32:64]
    gy = g[:, 64:96]
    gz = g[:, 96:128]
    gdot = gx * rx + gy * ry + gz * rz
    common = g0 * inv_r * inv_r2 + 2.0 * gdot * inv_r2 * inv_r2
    dfx = gx * inv_r2 - rx * common
    dfy = gy * inv_r2 - ry * common
    dfz = gz * inv_r2 - rz * common

    fex[...] = -dfx
    fey[...] = -dfy
    fez[...] = -dfz
    fself[:, 0:1] = jnp.sum(dfx, axis=1, keepdims=True)
    fself[:, 1:2] = jnp.sum(dfy, axis=1, keepdims=True)
    fself[:, 2:3] = jnp.sum(dfz, axis=1, keepdims=True)
    fself[:, 3:8] = jnp.zeros((B, 5), jnp.float32)


def _tc_dense(nbx, nby, nbz, cc, at, avg, istd, ws):
    B = BATOMS
    row = lambda i: (i, 0)
    fixed = lambda i: (0, 0)
    full = lambda shape: pl.BlockSpec(shape, fixed)
    in_specs = [
        pl.BlockSpec((B, N_NEI), row),
        pl.BlockSpec((B, N_NEI), row),
        pl.BlockSpec((B, N_NEI), row),
        pl.BlockSpec((B, 8), row),
        pl.BlockSpec((B, 8), row),
        full((8, 128)),
        full((8, 128)),
    ] + [full(w.shape) for w in ws]
    out_specs = [
        pl.BlockSpec((B, N_NEI), row),
        pl.BlockSpec((B, N_NEI), row),
        pl.BlockSpec((B, N_NEI), row),
        pl.BlockSpec((B, 8), row),
        pl.BlockSpec((1, 1), fixed),
    ]
    out_shape = [
        jax.ShapeDtypeStruct((N_ATOMS, N_NEI), jnp.float32),
        jax.ShapeDtypeStruct((N_ATOMS, N_NEI), jnp.float32),
        jax.ShapeDtypeStruct((N_ATOMS, N_NEI), jnp.float32),
        jax.ShapeDtypeStruct((N_ATOMS, 8), jnp.float32),
        jax.ShapeDtypeStruct((1, 1), jnp.float32),
    ]
    return pl.pallas_call(
        _dense_body,
        grid=(GRID,),
        in_specs=in_specs,
        out_specs=out_specs,
        out_shape=out_shape,
        compiler_params=pltpu.CompilerParams(
            dimension_semantics=("arbitrary",)),
    )(nbx, nby, nbz, cc, at, avg, istd, *ws)


def _pad2(a, rows, cols):
    return jnp.pad(a, ((0, rows - a.shape[0]), (0, cols - a.shape[1])))


def _group_cols(t):
    return jnp.concatenate([t[:, 0::4], t[:, 1::4], t[:, 2::4], t[:, 3::4]],
                           axis=1)


def kernel(coord, atype, nlist, t_avg, t_std,
           W0, b0, W1, b1, W2, b2, W3, b3, W4, b4, W5, b5):
    c0 = coord[0]
    cx = c0[:, 0]
    cy = c0[:, 1]
    cz = c0[:, 2]
    nidx = nlist.reshape(-1)

    nbx, nby, nbz = _sc_gather(cx, cy, cz, nidx)
    nbx = nbx.reshape(N_ATOMS, N_NEI)
    nby = nby.reshape(N_ATOMS, N_NEI)
    nbz = nbz.reshape(N_ATOMS, N_NEI)

    cc = jnp.pad(c0, ((0, 0), (0, 5)))
    at = jnp.broadcast_to(atype[0][:, None], (N_ATOMS, 8))
    avg = jnp.pad(_group_cols(t_avg), ((0, 6), (0, 0)))
    istd = jnp.pad(_group_cols(1.0 / t_std), ((0, 6), (0, 0)))

    w0g = jnp.concatenate([W0[0::4], W0[1::4], W0[2::4], W0[3::4]], axis=0)
    w0 = _pad2(w0g, 128, 256)
    w1 = _pad2(W1, 256, 128)
    w2 = _pad2(W2, 128, 64)
    w3 = _pad2(W3, 64, 32)
    w4 = _pad2(W4, 32, 16)
    w5r = _pad2(W5.T, 1, 16)
    b5p = _pad2(b5[None, :], 1, 8)
    ws = [w0, _pad2(b0[None, :], 1, 256),
          w1, _pad2(b1[None, :], 1, 128),
          w2, _pad2(b2[None, :], 1, 64),
          w3, _pad2(b3[None, :], 1, 32),
          w4, _pad2(b4[None, :], 1, 16),
          w5r, b5p,
          w1.T, w2.T, w3.T, w4.T, w0.T]

    fex, fey, fez, fself, ener = _tc_dense(nbx, nby, nbz, cc, at, avg, istd, ws)

    px, py, pz = _sc_scatter(fex.reshape(-1), fey.reshape(-1),
                             fez.reshape(-1), nidx)
    fx = fself[:, 0] + px[0, :N_ATOMS] + px[1, :N_ATOMS]
    fy = fself[:, 1] + py[0, :N_ATOMS] + py[1, :N_ATOMS]
    fz = fself[:, 2] + pz[0, :N_ATOMS] + pz[1, :N_ATOMS]
    force = jnp.stack([fx, fy, fz], axis=-1)[None]
    return ener.reshape(1), force


# speedup vs baseline: 9.4032x; 9.4032x over previous
import functools
import jax
import jax.numpy as jnp
from jax import lax
from jax.experimental import pallas as pl
from jax.experimental.pallas import tpu as pltpu
from jax.experimental.pallas import tpu_sc as plsc

N_ATOMS = 10000
N_NEI = 32
N_EDGES = N_ATOMS * N_NEI
NW = 32
EPT = N_EDGES // NW
NPAD = 10240
SLC = NPAD // 16
BATOMS = 1000
GRID = N_ATOMS // BATOMS

def _sc_mesh():
    return plsc.VectorSubcoreMesh(core_axis_name="c", subcore_axis_name="s",
                                  num_cores=2, num_subcores=16)


def _gather_body(cx, cy, cz, nidx, ox, oy, oz, idx_v, vx, vy, vz, sem):
    wid = lax.axis_index("s") * 2 + lax.axis_index("c")
    base = wid * EPT
    pltpu.sync_copy(nidx.at[pl.ds(base, EPT)], idx_v)
    pltpu.async_copy(cx.at[idx_v], vx, sem).wait()
    pltpu.async_copy(cy.at[idx_v], vy, sem).wait()
    pltpu.async_copy(cz.at[idx_v], vz, sem).wait()
    pltpu.sync_copy(vx, ox.at[pl.ds(base, EPT)])
    pltpu.sync_copy(vy, oy.at[pl.ds(base, EPT)])
    pltpu.sync_copy(vz, oz.at[pl.ds(base, EPT)])


def _sc_gather(cx, cy, cz, nidx):
    return pl.kernel(
        _gather_body,
        out_type=[jax.ShapeDtypeStruct((N_EDGES,), jnp.float32)] * 3,
        mesh=_sc_mesh(),
        scratch_types=[
            pltpu.VMEM((EPT,), jnp.int32),
            pltpu.VMEM((EPT,), jnp.float32),
            pltpu.VMEM((EPT,), jnp.float32),
            pltpu.VMEM((EPT,), jnp.float32),
            pltpu.SemaphoreType.DMA,
        ],
    )(cx, cy, cz, nidx)


def _scatter_body(fex, fey, fez, nidx, px, py, pz,
                  idx_v, val_v, zbuf, accx, accy, accz, sem):
    c = lax.axis_index("c")
    s = lax.axis_index("s")
    wid = s * 2 + c
    base = wid * EPT

    def _z(i, carry):
        zbuf[pl.ds(i * 16, 16)] = jnp.zeros((16,), jnp.float32)
        return carry
    lax.fori_loop(0, SLC // 16, _z, 0)
    pltpu.sync_copy(zbuf, accx.at[pl.ds(s * SLC, SLC)])
    pltpu.sync_copy(zbuf, accy.at[pl.ds(s * SLC, SLC)])
    pltpu.sync_copy(zbuf, accz.at[pl.ds(s * SLC, SLC)])
    plsc.subcore_barrier()

    pltpu.sync_copy(nidx.at[pl.ds(base, EPT)], idx_v)
    pltpu.sync_copy(fex.at[pl.ds(base, EPT)], val_v)
    pltpu.sync_copy(val_v, accx.at[idx_v], add=True)
    pltpu.sync_copy(fey.at[pl.ds(base, EPT)], val_v)
    pltpu.sync_copy(val_v, accy.at[idx_v], add=True)
    pltpu.sync_copy(fez.at[pl.ds(base, EPT)], val_v)
    pltpu.sync_copy(val_v, accz.at[idx_v], add=True)
    plsc.subcore_barrier()

    pltpu.sync_copy(accx.at[pl.ds(s * SLC, SLC)], px.at[c, pl.ds(s * SLC, SLC)])
    pltpu.sync_copy(accy.at[pl.ds(s * SLC, SLC)], py.at[c, pl.ds(s * SLC, SLC)])
    pltpu.sync_copy(accz.at[pl.ds(s * SLC, SLC)], pz.at[c, pl.ds(s * SLC, SLC)])


def _sc_scatter(fex, fey, fez, nidx):
    return pl.kernel(
        _scatter_body,
        out_type=[jax.ShapeDtypeStruct((2, NPAD), jnp.float32)] * 3,
        mesh=_sc_mesh(),
        scratch_types=[
            pltpu.VMEM((EPT,), jnp.int32),
            pltpu.VMEM((EPT,), jnp.float32),
            pltpu.VMEM((SLC,), jnp.float32),
            pltpu.VMEM_SHARED((NPAD,), jnp.float32),
            pltpu.VMEM_SHARED((NPAD,), jnp.float32),
            pltpu.VMEM_SHARED((NPAD,), jnp.float32),
            pltpu.SemaphoreType.DMA,
        ],
    )(fex, fey, fez, nidx)


def _dense_body(nbx, nby, nbz, cc, at, avg, istd,
                w0, b0, w1, b1, w2, b2, w3, b3, w4, b4, w5r, b5,
                w1t, w2t, w3t, w4t, w0t,
                fex, fey, fez, fself, ener):
    i = pl.program_id(0)
    B = BATOMS
    dot = functools.partial(jnp.dot, precision=lax.Precision.HIGHEST,
                            preferred_element_type=jnp.float32)

    cxc = cc[:, 0:1]
    cyc = cc[:, 1:2]
    czc = cc[:, 2:3]
    rx = nbx[...] - cxc
    ry = nby[...] - cyc
    rz = nbz[...] - czc
    r2 = rx * rx + ry * ry + rz * rz + 1e-6
    inv_r2 = 1.0 / r2
    r = jnp.sqrt(r2)
    inv_r = 1.0 / r

    raw = jnp.concatenate([inv_r, rx * inv_r2, ry * inv_r2, rz * inv_r2],
                          axis=1)
    sel = jnp.broadcast_to(at[:, 0:1] == 0, (B, 128))
    avg_row = jnp.where(sel, jnp.broadcast_to(avg[0:1, :], (B, 128)),
                        jnp.broadcast_to(avg[1:2, :], (B, 128)))
    istd_row = jnp.where(sel, jnp.broadcast_to(istd[0:1, :], (B, 128)),
                         jnp.broadcast_to(istd[1:2, :], (B, 128)))
    sdesc = (raw - avg_row) * istd_row

    h0 = jnp.tanh(dot(sdesc, w0[...]) + b0[...])
    h1 = jnp.tanh(dot(h0, w1[...]) + b1[...])
    h2 = jnp.tanh(dot(h1, w2[...]) + b2[...])
    h3 = jnp.tanh(dot(h2, w3[...]) + b3[...])
    h4 = jnp.tanh(dot(h3, w4[...]) + b4[...])
    atom_e = jnp.sum(h4 * w5r[...], axis=1, keepdims=True) + b5[0:1, 0:1]

    @pl.when(i == 0)
    def _():
        ener[...] = jnp.zeros((1, 128), jnp.float32)
    ener[...] += jnp.broadcast_to(jnp.sum(atom_e).reshape(1, 1), (1, 128))

    d4 = (1.0 - h4 * h4) * w5r[...]
    d3 = dot(d4, w4t[...]) * (1.0 - h3 * h3)
    d2 = dot(d3, w3t[...]) * (1.0 - h2 * h2)
    d1 = dot(d2, w2t[...]) * (1.0 - h1 * h1)
    d0 = dot(d1, w1t[...]) * (1.0 - h0 * h0)
    g = dot(d0, w0t[...]) * istd_row

    g0 = g[:, 0:32]
    gx = g[:, 32:64]
    gy = g[:, 64:96]
    gz = g[:, 96:128]
    gdot = gx * rx + gy * ry + gz * rz
    common = g0 * inv_r * inv_r2 + 2.0 * gdot * inv_r2 * inv_r2
    dfx = gx * inv_r2 - rx * common
    dfy = gy * inv_r2 - ry * common
    dfz = gz * inv_r2 - rz * common

    fex[...] = -dfx
    fey[...] = -dfy
    fez[...] = -dfz
    fself[:, 0:1] = jnp.sum(dfx, axis=1, keepdims=True)
    fself[:, 1:2] = jnp.sum(dfy, axis=1, keepdims=True)
    fself[:, 2:3] = jnp.sum(dfz, axis=1, keepdims=True)
    fself[:, 3:8] = jnp.zeros((B, 5), jnp.float32)


def _tc_dense(nbx, nby, nbz, cc, at, avg, istd, ws):
    B = BATOMS
    row = lambda i: (i, 0)
    fixed = lambda i: (0, 0)
    full = lambda shape: pl.BlockSpec(shape, fixed)
    in_specs = [
        pl.BlockSpec((B, N_NEI), row),
        pl.BlockSpec((B, N_NEI), row),
        pl.BlockSpec((B, N_NEI), row),
        pl.BlockSpec((B, 8), row),
        pl.BlockSpec((B, 8), row),
        full((8, 128)),
        full((8, 128)),
    ] + [full(w.shape) for w in ws]
    out_specs = [
        pl.BlockSpec((B, N_NEI), row),
        pl.BlockSpec((B, N_NEI), row),
        pl.BlockSpec((B, N_NEI), row),
        pl.BlockSpec((B, 8), row),
        pl.BlockSpec((1, 128), fixed),
    ]
    out_shape = [
        jax.ShapeDtypeStruct((N_ATOMS, N_NEI), jnp.float32),
        jax.ShapeDtypeStruct((N_ATOMS, N_NEI), jnp.float32),
        jax.ShapeDtypeStruct((N_ATOMS, N_NEI), jnp.float32),
        jax.ShapeDtypeStruct((N_ATOMS, 8), jnp.float32),
        jax.ShapeDtypeStruct((1, 128), jnp.float32),
    ]
    return pl.pallas_call(
        _dense_body,
        grid=(GRID,),
        in_specs=in_specs,
        out_specs=out_specs,
        out_shape=out_shape,
        compiler_params=pltpu.CompilerParams(
            dimension_semantics=("arbitrary",)),
    )(nbx, nby, nbz, cc, at, avg, istd, *ws)


def _pad2(a, rows, cols):
    return jnp.pad(a, ((0, rows - a.shape[0]), (0, cols - a.shape[1])))


def _group_cols(t):
    return jnp.concatenate([t[:, 0::4], t[:, 1::4], t[:, 2::4], t[:, 3::4]],
                           axis=1)


def kernel(coord, atype, nlist, t_avg, t_std,
           W0, b0, W1, b1, W2, b2, W3, b3, W4, b4, W5, b5):
    c0 = coord[0]
    cx = c0[:, 0]
    cy = c0[:, 1]
    cz = c0[:, 2]
    nidx = nlist.reshape(-1)

    nbx, nby, nbz = _sc_gather(cx, cy, cz, nidx)
    nbx = nbx.reshape(N_ATOMS, N_NEI)
    nby = nby.reshape(N_ATOMS, N_NEI)
    nbz = nbz.reshape(N_ATOMS, N_NEI)

    cc = jnp.pad(c0, ((0, 0), (0, 5)))
    at = jnp.broadcast_to(atype[0][:, None], (N_ATOMS, 8))
    avg = jnp.pad(_group_cols(t_avg), ((0, 6), (0, 0)))
    istd = jnp.pad(_group_cols(1.0 / t_std), ((0, 6), (0, 0)))

    w0g = jnp.concatenate([W0[0::4], W0[1::4], W0[2::4], W0[3::4]], axis=0)
    w0 = _pad2(w0g, 128, 256)
    w1 = _pad2(W1, 256, 128)
    w2 = _pad2(W2, 128, 64)
    w3 = _pad2(W3, 64, 32)
    w4 = _pad2(W4, 32, 16)
    w5r = _pad2(W5.T, 1, 16)
    b5p = _pad2(b5[None, :], 1, 8)
    ws = [w0, _pad2(b0[None, :], 1, 256),
          w1, _pad2(b1[None, :], 1, 128),
          w2, _pad2(b2[None, :], 1, 64),
          w3, _pad2(b3[None, :], 1, 32),
          w4, _pad2(b4[None, :], 1, 16),
          w5r, b5p,
          w1.T, w2.T, w3.T, w4.T, w0.T]

    fex, fey, fez, fself, ener = _tc_dense(nbx, nby, nbz, cc, at, avg, istd, ws)

    px, py, pz = _sc_scatter(fex.reshape(-1), fey.reshape(-1),
                             fez.reshape(-1), nidx)
    fx = fself[:, 0] + px[0, :N_ATOMS] + px[1, :N_ATOMS]
    fy = fself[:, 1] + py[0, :N_ATOMS] + py[1, :N_ATOMS]
    fz = fself[:, 2] + pz[0, :N_ATOMS] + pz[1, :N_ATOMS]
    force = jnp.stack([fx, fy, fz], axis=-1)[None]
    return ener[0, 0:1], force


# speedup vs baseline: 11.9436x; 1.2702x over previous
import functools
import jax
import jax.numpy as jnp
from jax import lax
from jax.experimental import pallas as pl
from jax.experimental.pallas import tpu as pltpu
from jax.experimental.pallas import tpu_sc as plsc

N_ATOMS = 10000
N_NEI = 32
N_EDGES = N_ATOMS * N_NEI
NW = 32
EPT = N_EDGES // NW
NPAD = 10240
SLC = NPAD // 16
BATOMS = 1000
GRID = N_ATOMS // BATOMS

def _sc_mesh():
    return plsc.VectorSubcoreMesh(core_axis_name="c", subcore_axis_name="s",
                                  num_cores=2, num_subcores=16)


def _gather_body(cx, cy, cz, nidx, ox, oy, oz,
                 idx_v, vx, vy, vz, cxs, cys, czs, sem):
    s = lax.axis_index("s")
    wid = s * 2 + lax.axis_index("c")
    base = wid * EPT

    @pl.when(s == 0)
    def _():
        pltpu.sync_copy(cx, cxs)

    @pl.when(s == 1)
    def _():
        pltpu.sync_copy(cy, cys)

    @pl.when(s == 2)
    def _():
        pltpu.sync_copy(cz, czs)

    pltpu.sync_copy(nidx.at[pl.ds(base, EPT)], idx_v)
    plsc.subcore_barrier()
    a = pltpu.async_copy(cxs.at[idx_v], vx, sem)
    b = pltpu.async_copy(cys.at[idx_v], vy, sem)
    c = pltpu.async_copy(czs.at[idx_v], vz, sem)
    a.wait()
    pltpu.sync_copy(vx, ox.at[pl.ds(base, EPT)])
    b.wait()
    pltpu.sync_copy(vy, oy.at[pl.ds(base, EPT)])
    c.wait()
    pltpu.sync_copy(vz, oz.at[pl.ds(base, EPT)])


def _sc_gather(cx, cy, cz, nidx):
    return pl.kernel(
        _gather_body,
        out_type=[jax.ShapeDtypeStruct((N_EDGES,), jnp.float32)] * 3,
        mesh=_sc_mesh(),
        scratch_types=[
            pltpu.VMEM((EPT,), jnp.int32),
            pltpu.VMEM((EPT,), jnp.float32),
            pltpu.VMEM((EPT,), jnp.float32),
            pltpu.VMEM((EPT,), jnp.float32),
            pltpu.VMEM_SHARED((N_ATOMS,), jnp.float32),
            pltpu.VMEM_SHARED((N_ATOMS,), jnp.float32),
            pltpu.VMEM_SHARED((N_ATOMS,), jnp.float32),
            pltpu.SemaphoreType.DMA,
        ],
    )(cx, cy, cz, nidx)


def _scatter_body(fex, fey, fez, nidx, px, py, pz,
                  idx_v, val_v, zbuf, accx, accy, accz, sem):
    c = lax.axis_index("c")
    s = lax.axis_index("s")
    wid = s * 2 + c
    base = wid * EPT

    def _z(i, carry):
        zbuf[pl.ds(i * 16, 16)] = jnp.zeros((16,), jnp.float32)
        return carry
    lax.fori_loop(0, SLC // 16, _z, 0)
    pltpu.sync_copy(zbuf, accx.at[pl.ds(s * SLC, SLC)])
    pltpu.sync_copy(zbuf, accy.at[pl.ds(s * SLC, SLC)])
    pltpu.sync_copy(zbuf, accz.at[pl.ds(s * SLC, SLC)])
    plsc.subcore_barrier()

    pltpu.sync_copy(nidx.at[pl.ds(base, EPT)], idx_v)
    pltpu.sync_copy(fex.at[pl.ds(base, EPT)], val_v)
    pltpu.sync_copy(val_v, accx.at[idx_v], add=True)
    pltpu.sync_copy(fey.at[pl.ds(base, EPT)], val_v)
    pltpu.sync_copy(val_v, accy.at[idx_v], add=True)
    pltpu.sync_copy(fez.at[pl.ds(base, EPT)], val_v)
    pltpu.sync_copy(val_v, accz.at[idx_v], add=True)
    plsc.subcore_barrier()

    pltpu.sync_copy(accx.at[pl.ds(s * SLC, SLC)], px.at[c, pl.ds(s * SLC, SLC)])
    pltpu.sync_copy(accy.at[pl.ds(s * SLC, SLC)], py.at[c, pl.ds(s * SLC, SLC)])
    pltpu.sync_copy(accz.at[pl.ds(s * SLC, SLC)], pz.at[c, pl.ds(s * SLC, SLC)])


def _sc_scatter(fex, fey, fez, nidx):
    return pl.kernel(
        _scatter_body,
        out_type=[jax.ShapeDtypeStruct((2, NPAD), jnp.float32)] * 3,
        mesh=_sc_mesh(),
        scratch_types=[
            pltpu.VMEM((EPT,), jnp.int32),
            pltpu.VMEM((EPT,), jnp.float32),
            pltpu.VMEM((SLC,), jnp.float32),
            pltpu.VMEM_SHARED((NPAD,), jnp.float32),
            pltpu.VMEM_SHARED((NPAD,), jnp.float32),
            pltpu.VMEM_SHARED((NPAD,), jnp.float32),
            pltpu.SemaphoreType.DMA,
        ],
    )(fex, fey, fez, nidx)


def _dense_body(nbx, nby, nbz, cc, at, avg, istd,
                w0, b0, w1, b1, w2, b2, w3, b3, w4, b4, w5r, b5,
                w1t, w2t, w3t, w4t, w0t,
                fex, fey, fez, fself, ener):
    i = pl.program_id(0)
    B = BATOMS
    dot = functools.partial(jnp.dot, precision=lax.Precision.HIGHEST,
                            preferred_element_type=jnp.float32)

    cxc = cc[:, 0:1]
    cyc = cc[:, 1:2]
    czc = cc[:, 2:3]
    rx = nbx[...] - cxc
    ry = nby[...] - cyc
    rz = nbz[...] - czc
    r2 = rx * rx + ry * ry + rz * rz + 1e-6
    inv_r2 = 1.0 / r2
    r = jnp.sqrt(r2)
    inv_r = 1.0 / r

    raw = jnp.concatenate([inv_r, rx * inv_r2, ry * inv_r2, rz * inv_r2],
                          axis=1)
    sel = jnp.broadcast_to(at[:, 0:1] == 0, (B, 128))
    avg_row = jnp.where(sel, jnp.broadcast_to(avg[0:1, :], (B, 128)),
                        jnp.broadcast_to(avg[1:2, :], (B, 128)))
    istd_row = jnp.where(sel, jnp.broadcast_to(istd[0:1, :], (B, 128)),
                         jnp.broadcast_to(istd[1:2, :], (B, 128)))
    sdesc = (raw - avg_row) * istd_row

    h0 = jnp.tanh(dot(sdesc, w0[...]) + b0[...])
    h1 = jnp.tanh(dot(h0, w1[...]) + b1[...])
    h2 = jnp.tanh(dot(h1, w2[...]) + b2[...])
    h3 = jnp.tanh(dot(h2, w3[...]) + b3[...])
    h4 = jnp.tanh(dot(h3, w4[...]) + b4[...])
    atom_e = jnp.sum(h4 * w5r[...], axis=1, keepdims=True) + b5[0:1, 0:1]

    @pl.when(i == 0)
    def _():
        ener[...] = jnp.zeros((1, 128), jnp.float32)
    ener[...] += jnp.broadcast_to(jnp.sum(atom_e).reshape(1, 1), (1, 128))

    d4 = (1.0 - h4 * h4) * w5r[...]
    d3 = dot(d4, w4t[...]) * (1.0 - h3 * h3)
    d2 = dot(d3, w3t[...]) * (1.0 - h2 * h2)
    d1 = dot(d2, w2t[...]) * (1.0 - h1 * h1)
    d0 = dot(d1, w1t[...]) * (1.0 - h0 * h0)
    g = dot(d0, w0t[...]) * istd_row

    g0 = g[:, 0:32]
    gx = g[:, 32:64]
    gy = g[:, 64:96]
    gz = g[:, 96:128]
    gdot = gx * rx + gy * ry + gz * rz
    common = g0 * inv_r * inv_r2 + 2.0 * gdot * inv_r2 * inv_r2
    dfx = gx * inv_r2 - rx * common
    dfy = gy * inv_r2 - ry * common
    dfz = gz * inv_r2 - rz * common

    fex[...] = -dfx
    fey[...] = -dfy
    fez[...] = -dfz
    fself[:, 0:1] = jnp.sum(dfx, axis=1, keepdims=True)
    fself[:, 1:2] = jnp.sum(dfy, axis=1, keepdims=True)
    fself[:, 2:3] = jnp.sum(dfz, axis=1, keepdims=True)
    fself[:, 3:8] = jnp.zeros((B, 5), jnp.float32)


def _tc_dense(nbx, nby, nbz, cc, at, avg, istd, ws):
    B = BATOMS
    row = lambda i: (i, 0)
    fixed = lambda i: (0, 0)
    full = lambda shape: pl.BlockSpec(shape, fixed)
    in_specs = [
        pl.BlockSpec((B, N_NEI), row),
        pl.BlockSpec((B, N_NEI), row),
        pl.BlockSpec((B, N_NEI), row),
        pl.BlockSpec((B, 8), row),
        pl.BlockSpec((B, 8), row),
        full((8, 128)),
        full((8, 128)),
    ] + [full(w.shape) for w in ws]
    out_specs = [
        pl.BlockSpec((B, N_NEI), row),
        pl.BlockSpec((B, N_NEI), row),
        pl.BlockSpec((B, N_NEI), row),
        pl.BlockSpec((B, 8), row),
        pl.BlockSpec((1, 128), fixed),
    ]
    out_shape = [
        jax.ShapeDtypeStruct((N_ATOMS, N_NEI), jnp.float32),
        jax.ShapeDtypeStruct((N_ATOMS, N_NEI), jnp.float32),
        jax.ShapeDtypeStruct((N_ATOMS, N_NEI), jnp.float32),
        jax.ShapeDtypeStruct((N_ATOMS, 8), jnp.float32),
        jax.ShapeDtypeStruct((1, 128), jnp.float32),
    ]
    return pl.pallas_call(
        _dense_body,
        grid=(GRID,),
        in_specs=in_specs,
        out_specs=out_specs,
        out_shape=out_shape,
        compiler_params=pltpu.CompilerParams(
            dimension_semantics=("arbitrary",)),
    )(nbx, nby, nbz, cc, at, avg, istd, *ws)


def _pad2(a, rows, cols):
    return jnp.pad(a, ((0, rows - a.shape[0]), (0, cols - a.shape[1])))


def _group_cols(t):
    return jnp.concatenate([t[:, 0::4], t[:, 1::4], t[:, 2::4], t[:, 3::4]],
                           axis=1)


def kernel(coord, atype, nlist, t_avg, t_std,
           W0, b0, W1, b1, W2, b2, W3, b3, W4, b4, W5, b5):
    c0 = coord[0]
    cx = c0[:, 0]
    cy = c0[:, 1]
    cz = c0[:, 2]
    nidx = nlist.reshape(-1)

    nbx, nby, nbz = _sc_gather(cx, cy, cz, nidx)
    nbx = nbx.reshape(N_ATOMS, N_NEI)
    nby = nby.reshape(N_ATOMS, N_NEI)
    nbz = nbz.reshape(N_ATOMS, N_NEI)

    cc = jnp.pad(c0, ((0, 0), (0, 5)))
    at = jnp.broadcast_to(atype[0][:, None], (N_ATOMS, 8))
    avg = jnp.pad(_group_cols(t_avg), ((0, 6), (0, 0)))
    istd = jnp.pad(_group_cols(1.0 / t_std), ((0, 6), (0, 0)))

    w0g = jnp.concatenate([W0[0::4], W0[1::4], W0[2::4], W0[3::4]], axis=0)
    w0 = _pad2(w0g, 128, 256)
    w1 = _pad2(W1, 256, 128)
    w2 = _pad2(W2, 128, 64)
    w3 = _pad2(W3, 64, 32)
    w4 = _pad2(W4, 32, 16)
    w5r = _pad2(W5.T, 1, 16)
    b5p = _pad2(b5[None, :], 1, 8)
    ws = [w0, _pad2(b0[None, :], 1, 256),
          w1, _pad2(b1[None, :], 1, 128),
          w2, _pad2(b2[None, :], 1, 64),
          w3, _pad2(b3[None, :], 1, 32),
          w4, _pad2(b4[None, :], 1, 16),
          w5r, b5p,
          w1.T, w2.T, w3.T, w4.T, w0.T]

    fex, fey, fez, fself, ener = _tc_dense(nbx, nby, nbz, cc, at, avg, istd, ws)

    px, py, pz = _sc_scatter(fex.reshape(-1), fey.reshape(-1),
                             fez.reshape(-1), nidx)
    fx = fself[:, 0] + px[0, :N_ATOMS] + px[1, :N_ATOMS]
    fy = fself[:, 1] + py[0, :N_ATOMS] + py[1, :N_ATOMS]
    fz = fself[:, 2] + pz[0, :N_ATOMS] + pz[1, :N_ATOMS]
    force = jnp.stack([fx, fy, fz], axis=-1)[None]
    return ener[0, 0:1], force


# speedup vs baseline: 19.6616x; 1.6462x over previous
import functools
import jax
import jax.numpy as jnp
from jax import lax
from jax.experimental import pallas as pl
from jax.experimental.pallas import tpu as pltpu
from jax.experimental.pallas import tpu_sc as plsc

N_ATOMS = 10000
N_NEI = 32
N_EDGES = N_ATOMS * N_NEI
NW = 32
EPT = N_EDGES // NW
NPAD = 10240
SLC = NPAD // 16
BATOMS = 1000
GRID = N_ATOMS // BATOMS

def _sc_mesh():
    return plsc.VectorSubcoreMesh(core_axis_name="c", subcore_axis_name="s",
                                  num_cores=2, num_subcores=16)


def _gather_body(cx, cy, cz, nidx, ox, oy, oz,
                 idx_v, vx, vy, vz, cxs, cys, czs, sem):
    s = lax.axis_index("s")
    wid = s * 2 + lax.axis_index("c")
    base = wid * EPT

    @pl.when(s == 0)
    def _():
        pltpu.sync_copy(cx, cxs)

    @pl.when(s == 1)
    def _():
        pltpu.sync_copy(cy, cys)

    @pl.when(s == 2)
    def _():
        pltpu.sync_copy(cz, czs)

    pltpu.sync_copy(nidx.at[pl.ds(base, EPT)], idx_v)
    plsc.subcore_barrier()
    a = pltpu.async_copy(cxs.at[idx_v], vx, sem)
    b = pltpu.async_copy(cys.at[idx_v], vy, sem)
    c = pltpu.async_copy(czs.at[idx_v], vz, sem)
    a.wait()
    pltpu.sync_copy(vx, ox.at[pl.ds(base, EPT)])
    b.wait()
    pltpu.sync_copy(vy, oy.at[pl.ds(base, EPT)])
    c.wait()
    pltpu.sync_copy(vz, oz.at[pl.ds(base, EPT)])


def _sc_gather(cx, cy, cz, nidx):
    return pl.kernel(
        _gather_body,
        out_type=[jax.ShapeDtypeStruct((N_EDGES,), jnp.float32)] * 3,
        mesh=_sc_mesh(),
        scratch_types=[
            pltpu.VMEM((EPT,), jnp.int32),
            pltpu.VMEM((EPT,), jnp.float32),
            pltpu.VMEM((EPT,), jnp.float32),
            pltpu.VMEM((EPT,), jnp.float32),
            pltpu.VMEM_SHARED((N_ATOMS,), jnp.float32),
            pltpu.VMEM_SHARED((N_ATOMS,), jnp.float32),
            pltpu.VMEM_SHARED((N_ATOMS,), jnp.float32),
            pltpu.SemaphoreType.DMA,
        ],
    )(cx, cy, cz, nidx)


def _scatter_body(fex, fey, fez, nidx, px, py, pz,
                  idx_v, val_v, zbuf, accx, accy, accz, sem):
    c = lax.axis_index("c")
    s = lax.axis_index("s")
    wid = s * 2 + c
    base = wid * EPT

    def _z(i, carry):
        zbuf[pl.ds(i * 16, 16)] = jnp.zeros((16,), jnp.float32)
        return carry
    lax.fori_loop(0, SLC // 16, _z, 0)
    pltpu.sync_copy(zbuf, accx.at[pl.ds(s * SLC, SLC)])
    pltpu.sync_copy(zbuf, accy.at[pl.ds(s * SLC, SLC)])
    pltpu.sync_copy(zbuf, accz.at[pl.ds(s * SLC, SLC)])
    plsc.subcore_barrier()

    pltpu.sync_copy(nidx.at[pl.ds(base, EPT)], idx_v)
    pltpu.sync_copy(fex.at[pl.ds(base, EPT)], val_v)
    pltpu.sync_copy(val_v, accx.at[idx_v], add=True)
    pltpu.sync_copy(fey.at[pl.ds(base, EPT)], val_v)
    pltpu.sync_copy(val_v, accy.at[idx_v], add=True)
    pltpu.sync_copy(fez.at[pl.ds(base, EPT)], val_v)
    pltpu.sync_copy(val_v, accz.at[idx_v], add=True)
    plsc.subcore_barrier()

    pltpu.sync_copy(accx.at[pl.ds(s * SLC, SLC)], px.at[c, pl.ds(s * SLC, SLC)])
    pltpu.sync_copy(accy.at[pl.ds(s * SLC, SLC)], py.at[c, pl.ds(s * SLC, SLC)])
    pltpu.sync_copy(accz.at[pl.ds(s * SLC, SLC)], pz.at[c, pl.ds(s * SLC, SLC)])


def _sc_scatter(fex, fey, fez, nidx):
    return pl.kernel(
        _scatter_body,
        out_type=[jax.ShapeDtypeStruct((2, NPAD), jnp.float32)] * 3,
        mesh=_sc_mesh(),
        scratch_types=[
            pltpu.VMEM((EPT,), jnp.int32),
            pltpu.VMEM((EPT,), jnp.float32),
            pltpu.VMEM((SLC,), jnp.float32),
            pltpu.VMEM_SHARED((NPAD,), jnp.float32),
            pltpu.VMEM_SHARED((NPAD,), jnp.float32),
            pltpu.VMEM_SHARED((NPAD,), jnp.float32),
            pltpu.SemaphoreType.DMA,
        ],
    )(fex, fey, fez, nidx)


def _dense_body(nbx, nby, nbz, cc, at, avg, istd,
                w0, b0, w1, b1, w2, b2, w3, b3, w4, b4, w5r, b5,
                w1t, w2t, w3t, w4t, w0t,
                fex, fey, fez, fself, ener):
    i = pl.program_id(0)
    B = BATOMS
    dot = functools.partial(jnp.dot, precision=lax.Precision.DEFAULT,
                            preferred_element_type=jnp.float32)

    cxc = cc[:, 0:1]
    cyc = cc[:, 1:2]
    czc = cc[:, 2:3]
    rx = nbx[...] - cxc
    ry = nby[...] - cyc
    rz = nbz[...] - czc
    r2 = rx * rx + ry * ry + rz * rz + 1e-6
    inv_r2 = 1.0 / r2
    r = jnp.sqrt(r2)
    inv_r = 1.0 / r

    raw = jnp.concatenate([inv_r, rx * inv_r2, ry * inv_r2, rz * inv_r2],
                          axis=1)
    sel = jnp.broadcast_to(at[:, 0:1] == 0, (B, 128))
    avg_row = jnp.where(sel, jnp.broadcast_to(avg[0:1, :], (B, 128)),
                        jnp.broadcast_to(avg[1:2, :], (B, 128)))
    istd_row = jnp.where(sel, jnp.broadcast_to(istd[0:1, :], (B, 128)),
                         jnp.broadcast_to(istd[1:2, :], (B, 128)))
    sdesc = (raw - avg_row) * istd_row

    h0 = jnp.tanh(dot(sdesc, w0[...]) + b0[...])
    h1 = jnp.tanh(dot(h0, w1[...]) + b1[...])
    h2 = jnp.tanh(dot(h1, w2[...]) + b2[...])
    h3 = jnp.tanh(dot(h2, w3[...]) + b3[...])
    h4 = jnp.tanh(dot(h3, w4[...]) + b4[...])
    atom_e = jnp.sum(h4 * w5r[...], axis=1, keepdims=True) + b5[0:1, 0:1]

    @pl.when(i == 0)
    def _():
        ener[...] = jnp.zeros((1, 128), jnp.float32)
    ener[...] += jnp.broadcast_to(jnp.sum(atom_e).reshape(1, 1), (1, 128))

    d4 = (1.0 - h4 * h4) * w5r[...]
    d3 = dot(d4, w4t[...]) * (1.0 - h3 * h3)
    d2 = dot(d3, w3t[...]) * (1.0 - h2 * h2)
    d1 = dot(d2, w2t[...]) * (1.0 - h1 * h1)
    d0 = dot(d1, w1t[...]) * (1.0 - h0 * h0)
    g = dot(d0, w0t[...]) * istd_row

    g0 = g[:, 0:32]
    gx = g[:, 32:64]
    gy = g[:, 64:96]
    gz = g[:, 96:128]
    gdot = gx * rx + gy * ry + gz * rz
    common = g0 * inv_r * inv_r2 + 2.0 * gdot * inv_r2 * inv_r2
    dfx = gx * inv_r2 - rx * common
    dfy = gy * inv_r2 - ry * common
    dfz = gz * inv_r2 - rz * common

    fex[...] = -dfx
    fey[...] = -dfy
    fez[...] = -dfz
    fself[:, 0:1] = jnp.sum(dfx, axis=1, keepdims=True)
    fself[:, 1:2] = jnp.sum(dfy, axis=1, keepdims=True)
    fself[:, 2:3] = jnp.sum(dfz, axis=1, keepdims=True)
    fself[:, 3:8] = jnp.zeros((B, 5), jnp.float32)


def _tc_dense(nbx, nby, nbz, cc, at, avg, istd, ws):
    B = BATOMS
    row = lambda i: (i, 0)
    fixed = lambda i: (0, 0)
    full = lambda shape: pl.BlockSpec(shape, fixed)
    in_specs = [
        pl.BlockSpec((B, N_NEI), row),
        pl.BlockSpec((B, N_NEI), row),
        pl.BlockSpec((B, N_NEI), row),
        pl.BlockSpec((B, 8), row),
        pl.BlockSpec((B, 8), row),
        full((8, 128)),
        full((8, 128)),
    ] + [full(w.shape) for w in ws]
    out_specs = [
        pl.BlockSpec((B, N_NEI), row),
        pl.BlockSpec((B, N_NEI), row),
        pl.BlockSpec((B, N_NEI), row),
        pl.BlockSpec((B, 8), row),
        pl.BlockSpec((1, 128), fixed),
    ]
    out_shape = [
        jax.ShapeDtypeStruct((N_ATOMS, N_NEI), jnp.float32),
        jax.ShapeDtypeStruct((N_ATOMS, N_NEI), jnp.float32),
        jax.ShapeDtypeStruct((N_ATOMS, N_NEI), jnp.float32),
        jax.ShapeDtypeStruct((N_ATOMS, 8), jnp.float32),
        jax.ShapeDtypeStruct((1, 128), jnp.float32),
    ]
    return pl.pallas_call(
        _dense_body,
        grid=(GRID,),
        in_specs=in_specs,
        out_specs=out_specs,
        out_shape=out_shape,
        compiler_params=pltpu.CompilerParams(
            dimension_semantics=("arbitrary",)),
    )(nbx, nby, nbz, cc, at, avg, istd, *ws)


def _pad2(a, rows, cols):
    return jnp.pad(a, ((0, rows - a.shape[0]), (0, cols - a.shape[1])))


def _group_cols(t):
    return jnp.concatenate([t[:, 0::4], t[:, 1::4], t[:, 2::4], t[:, 3::4]],
                           axis=1)


def kernel(coord, atype, nlist, t_avg, t_std,
           W0, b0, W1, b1, W2, b2, W3, b3, W4, b4, W5, b5):
    c0 = coord[0]
    cx = c0[:, 0]
    cy = c0[:, 1]
    cz = c0[:, 2]
    nidx = nlist.reshape(-1)

    nbx, nby, nbz = _sc_gather(cx, cy, cz, nidx)
    nbx = nbx.reshape(N_ATOMS, N_NEI)
    nby = nby.reshape(N_ATOMS, N_NEI)
    nbz = nbz.reshape(N_ATOMS, N_NEI)

    cc = jnp.pad(c0, ((0, 0), (0, 5)))
    at = jnp.broadcast_to(atype[0][:, None], (N_ATOMS, 8))
    avg = jnp.pad(_group_cols(t_avg), ((0, 6), (0, 0)))
    istd = jnp.pad(_group_cols(1.0 / t_std), ((0, 6), (0, 0)))

    w0g = jnp.concatenate([W0[0::4], W0[1::4], W0[2::4], W0[3::4]], axis=0)
    w0 = _pad2(w0g, 128, 256)
    w1 = _pad2(W1, 256, 128)
    w2 = _pad2(W2, 128, 64)
    w3 = _pad2(W3, 64, 32)
    w4 = _pad2(W4, 32, 16)
    w5r = _pad2(W5.T, 1, 16)
    b5p = _pad2(b5[None, :], 1, 8)
    ws = [w0, _pad2(b0[None, :], 1, 256),
          w1, _pad2(b1[None, :], 1, 128),
          w2, _pad2(b2[None, :], 1, 64),
          w3, _pad2(b3[None, :], 1, 32),
          w4, _pad2(b4[None, :], 1, 16),
          w5r, b5p,
          w1.T, w2.T, w3.T, w4.T, w0.T]

    fex, fey, fez, fself, ener = _tc_dense(nbx, nby, nbz, cc, at, avg, istd, ws)

    px, py, pz = _sc_scatter(fex.reshape(-1), fey.reshape(-1),
                             fez.reshape(-1), nidx)
    fx = fself[:, 0] + px[0, :N_ATOMS] + px[1, :N_ATOMS]
    fy = fself[:, 1] + py[0, :N_ATOMS] + py[1, :N_ATOMS]
    fz = fself[:, 2] + pz[0, :N_ATOMS] + pz[1, :N_ATOMS]
    force = jnp.stack([fx, fy, fz], axis=-1)[None]
    return ener[0, 0:1], force


# speedup vs baseline: 20.9756x; 1.0668x over previous
import functools
import jax
import jax.numpy as jnp
from jax import lax
from jax.experimental import pallas as pl
from jax.experimental.pallas import tpu as pltpu
from jax.experimental.pallas import tpu_sc as plsc

N_ATOMS = 10000
N_NEI = 32
N_EDGES = N_ATOMS * N_NEI
NW = 32
EPT = N_EDGES // NW
NPAD = 10240
SLC = NPAD // 16
BATOMS = 1000
GRID = N_ATOMS // BATOMS

def _sc_mesh():
    return plsc.VectorSubcoreMesh(core_axis_name="c", subcore_axis_name="s",
                                  num_cores=2, num_subcores=16)


def _gather_body(cx, cy, cz, nidx, ox, oy, oz,
                 idx_v, vx, vy, vz, cxs, cys, czs, sem):
    s = lax.axis_index("s")
    wid = s * 2 + lax.axis_index("c")
    base = wid * EPT

    @pl.when(s == 0)
    def _():
        pltpu.sync_copy(cx, cxs)

    @pl.when(s == 1)
    def _():
        pltpu.sync_copy(cy, cys)

    @pl.when(s == 2)
    def _():
        pltpu.sync_copy(cz, czs)

    pltpu.sync_copy(nidx.at[pl.ds(base, EPT)], idx_v)
    plsc.subcore_barrier()
    a = pltpu.async_copy(cxs.at[idx_v], vx, sem)
    b = pltpu.async_copy(cys.at[idx_v], vy, sem)
    c = pltpu.async_copy(czs.at[idx_v], vz, sem)
    a.wait()
    pltpu.sync_copy(vx, ox.at[pl.ds(base, EPT)])
    b.wait()
    pltpu.sync_copy(vy, oy.at[pl.ds(base, EPT)])
    c.wait()
    pltpu.sync_copy(vz, oz.at[pl.ds(base, EPT)])


def _sc_gather(cx, cy, cz, nidx):
    return pl.kernel(
        _gather_body,
        out_type=[jax.ShapeDtypeStruct((N_EDGES,), jnp.float32)] * 3,
        mesh=_sc_mesh(),
        scratch_types=[
            pltpu.VMEM((EPT,), jnp.int32),
            pltpu.VMEM((EPT,), jnp.float32),
            pltpu.VMEM((EPT,), jnp.float32),
            pltpu.VMEM((EPT,), jnp.float32),
            pltpu.VMEM_SHARED((N_ATOMS,), jnp.float32),
            pltpu.VMEM_SHARED((N_ATOMS,), jnp.float32),
            pltpu.VMEM_SHARED((N_ATOMS,), jnp.float32),
            pltpu.SemaphoreType.DMA,
        ],
    )(cx, cy, cz, nidx)


def _scatter_body(fex, fey, fez, nidx, px, py, pz,
                  idx_v, val_v, zbuf, accx, accy, accz, sem):
    c = lax.axis_index("c")
    s = lax.axis_index("s")
    wid = s * 2 + c
    base = wid * EPT

    def _z(i, carry):
        zbuf[pl.ds(i * 16, 16)] = jnp.zeros((16,), jnp.float32)
        return carry
    lax.fori_loop(0, SLC // 16, _z, 0)
    pltpu.sync_copy(zbuf, accx.at[pl.ds(s * SLC, SLC)])
    pltpu.sync_copy(zbuf, accy.at[pl.ds(s * SLC, SLC)])
    pltpu.sync_copy(zbuf, accz.at[pl.ds(s * SLC, SLC)])
    plsc.subcore_barrier()

    pltpu.sync_copy(nidx.at[pl.ds(base, EPT)], idx_v)
    pltpu.sync_copy(fex.at[pl.ds(base, EPT)], val_v)
    pltpu.sync_copy(val_v, accx.at[idx_v], add=True)
    pltpu.sync_copy(fey.at[pl.ds(base, EPT)], val_v)
    pltpu.sync_copy(val_v, accy.at[idx_v], add=True)
    pltpu.sync_copy(fez.at[pl.ds(base, EPT)], val_v)
    pltpu.sync_copy(val_v, accz.at[idx_v], add=True)
    plsc.subcore_barrier()

    pltpu.sync_copy(accx.at[pl.ds(s * SLC, SLC)], px.at[c, pl.ds(s * SLC, SLC)])
    pltpu.sync_copy(accy.at[pl.ds(s * SLC, SLC)], py.at[c, pl.ds(s * SLC, SLC)])
    pltpu.sync_copy(accz.at[pl.ds(s * SLC, SLC)], pz.at[c, pl.ds(s * SLC, SLC)])


def _sc_scatter(fex, fey, fez, nidx):
    return pl.kernel(
        _scatter_body,
        out_type=[jax.ShapeDtypeStruct((2, NPAD), jnp.float32)] * 3,
        mesh=_sc_mesh(),
        scratch_types=[
            pltpu.VMEM((EPT,), jnp.int32),
            pltpu.VMEM((EPT,), jnp.float32),
            pltpu.VMEM((SLC,), jnp.float32),
            pltpu.VMEM_SHARED((NPAD,), jnp.float32),
            pltpu.VMEM_SHARED((NPAD,), jnp.float32),
            pltpu.VMEM_SHARED((NPAD,), jnp.float32),
            pltpu.SemaphoreType.DMA,
        ],
    )(fex, fey, fez, nidx)


def _dense_body(nbx, nby, nbz, cc, at, avg, istd,
                w0, b0, w1, b1, w2, b2, w3, b3, w4, b4, w5r, b5,
                fex, fey, fez, fself, ener):
    i = pl.program_id(0)
    B = BATOMS
    dot = functools.partial(jnp.dot, precision=lax.Precision.DEFAULT,
                            preferred_element_type=jnp.float32)
    dotT = functools.partial(lax.dot_general,
                             dimension_numbers=(((1,), (1,)), ((), ())),
                             precision=lax.Precision.DEFAULT,
                             preferred_element_type=jnp.float32)

    cxc = cc[:, 0:1]
    cyc = cc[:, 1:2]
    czc = cc[:, 2:3]
    rx = nbx[...] - cxc
    ry = nby[...] - cyc
    rz = nbz[...] - czc
    r2 = rx * rx + ry * ry + rz * rz + 1e-6
    inv_r2 = 1.0 / r2
    r = jnp.sqrt(r2)
    inv_r = 1.0 / r

    raw = jnp.concatenate([inv_r, rx * inv_r2, ry * inv_r2, rz * inv_r2],
                          axis=1)
    sel = jnp.broadcast_to(at[:, 0:1] == 0, (B, 128))
    avg_row = jnp.where(sel, jnp.broadcast_to(avg[0:1, :], (B, 128)),
                        jnp.broadcast_to(avg[1:2, :], (B, 128)))
    istd_row = jnp.where(sel, jnp.broadcast_to(istd[0:1, :], (B, 128)),
                         jnp.broadcast_to(istd[1:2, :], (B, 128)))
    sdesc = (raw - avg_row) * istd_row

    h0 = jnp.tanh(dot(sdesc, w0[...]) + b0[...])
    h1 = jnp.tanh(dot(h0, w1[...]) + b1[...])
    h2 = jnp.tanh(dot(h1, w2[...]) + b2[...])
    h3 = jnp.tanh(dot(h2, w3[...]) + b3[...])
    h4 = jnp.tanh(dot(h3, w4[...]) + b4[...])
    atom_e = jnp.sum(h4 * w5r[...], axis=1, keepdims=True) + b5[0:1, 0:1]

    @pl.when(i == 0)
    def _():
        ener[...] = jnp.zeros((1, 128), jnp.float32)
    ener[...] += jnp.broadcast_to(jnp.sum(atom_e).reshape(1, 1), (1, 128))

    d4 = (1.0 - h4 * h4) * w5r[...]
    d3 = dotT(d4, w4[...]) * (1.0 - h3 * h3)
    d2 = dotT(d3, w3[...]) * (1.0 - h2 * h2)
    d1 = dotT(d2, w2[...]) * (1.0 - h1 * h1)
    d0 = dotT(d1, w1[...]) * (1.0 - h0 * h0)
    g = dotT(d0, w0[...]) * istd_row

    g0 = g[:, 0:32]
    gx = g[:, 32:64]
    gy = g[:, 64:96]
    gz = g[:, 96:128]
    gdot = gx * rx + gy * ry + gz * rz
    common = g0 * inv_r * inv_r2 + 2.0 * gdot * inv_r2 * inv_r2
    dfx = gx * inv_r2 - rx * common
    dfy = gy * inv_r2 - ry * common
    dfz = gz * inv_r2 - rz * common

    fex[...] = -dfx
    fey[...] = -dfy
    fez[...] = -dfz
    fself[:, 0:1] = jnp.sum(dfx, axis=1, keepdims=True)
    fself[:, 1:2] = jnp.sum(dfy, axis=1, keepdims=True)
    fself[:, 2:3] = jnp.sum(dfz, axis=1, keepdims=True)
    fself[:, 3:8] = jnp.zeros((B, 5), jnp.float32)


def _tc_dense(nbx, nby, nbz, cc, at, avg, istd, ws):
    B = BATOMS
    row = lambda i: (i, 0)
    fixed = lambda i: (0, 0)
    full = lambda shape: pl.BlockSpec(shape, fixed)
    in_specs = [
        pl.BlockSpec((B, N_NEI), row),
        pl.BlockSpec((B, N_NEI), row),
        pl.BlockSpec((B, N_NEI), row),
        pl.BlockSpec((B, 8), row),
        pl.BlockSpec((B, 8), row),
        full((8, 128)),
        full((8, 128)),
    ] + [full(w.shape) for w in ws]
    out_specs = [
        pl.BlockSpec((B, N_NEI), row),
        pl.BlockSpec((B, N_NEI), row),
        pl.BlockSpec((B, N_NEI), row),
        pl.BlockSpec((B, 8), row),
        pl.BlockSpec((1, 128), fixed),
    ]
    out_shape = [
        jax.ShapeDtypeStruct((N_ATOMS, N_NEI), jnp.float32),
        jax.ShapeDtypeStruct((N_ATOMS, N_NEI), jnp.float32),
        jax.ShapeDtypeStruct((N_ATOMS, N_NEI), jnp.float32),
        jax.ShapeDtypeStruct((N_ATOMS, 8), jnp.float32),
        jax.ShapeDtypeStruct((1, 128), jnp.float32),
    ]
    return pl.pallas_call(
        _dense_body,
        grid=(GRID,),
        in_specs=in_specs,
        out_specs=out_specs,
        out_shape=out_shape,
        compiler_params=pltpu.CompilerParams(
            dimension_semantics=("arbitrary",)),
    )(nbx, nby, nbz, cc, at, avg, istd, *ws)


def _pad2(a, rows, cols):
    return jnp.pad(a, ((0, rows - a.shape[0]), (0, cols - a.shape[1])))


def _group_cols(t):
    return jnp.concatenate([t[:, 0::4], t[:, 1::4], t[:, 2::4], t[:, 3::4]],
                           axis=1)


def kernel(coord, atype, nlist, t_avg, t_std,
           W0, b0, W1, b1, W2, b2, W3, b3, W4, b4, W5, b5):
    c0 = coord[0]
    cx = c0[:, 0]
    cy = c0[:, 1]
    cz = c0[:, 2]
    nidx = nlist.reshape(-1)

    nbx, nby, nbz = _sc_gather(cx, cy, cz, nidx)
    nbx = nbx.reshape(N_ATOMS, N_NEI)
    nby = nby.reshape(N_ATOMS, N_NEI)
    nbz = nbz.reshape(N_ATOMS, N_NEI)

    cc = jnp.pad(c0, ((0, 0), (0, 5)))
    at = jnp.broadcast_to(atype[0][:, None], (N_ATOMS, 8))
    avg = jnp.pad(_group_cols(t_avg), ((0, 6), (0, 0)))
    istd = jnp.pad(_group_cols(1.0 / t_std), ((0, 6), (0, 0)))

    w0g = W0.reshape(32, 4, 240).transpose(1, 0, 2).reshape(128, 240)
    w0 = _pad2(w0g, 128, 256)
    w1 = _pad2(W1, 256, 128)
    w2 = _pad2(W2, 128, 64)
    w3 = _pad2(W3, 64, 32)
    w4 = _pad2(W4, 32, 16)
    w5r = _pad2(W5.T, 1, 16)
    b5p = _pad2(b5[None, :], 1, 8)
    ws = [w0, _pad2(b0[None, :], 1, 256),
          w1, _pad2(b1[None, :], 1, 128),
          w2, _pad2(b2[None, :], 1, 64),
          w3, _pad2(b3[None, :], 1, 32),
          w4, _pad2(b4[None, :], 1, 16),
          w5r, b5p]

    fex, fey, fez, fself, ener = _tc_dense(nbx, nby, nbz, cc, at, avg, istd, ws)

    px, py, pz = _sc_scatter(fex.reshape(-1), fey.reshape(-1),
                             fez.reshape(-1), nidx)
    fx = fself[:, 0] + px[0, :N_ATOMS] + px[1, :N_ATOMS]
    fy = fself[:, 1] + py[0, :N_ATOMS] + py[1, :N_ATOMS]
    fz = fself[:, 2] + pz[0, :N_ATOMS] + pz[1, :N_ATOMS]
    force = jnp.stack([fx, fy, fz], axis=-1)[None]
    return ener[0, 0:1], force
